# baseline (speedup 1.0000x reference)
import functools

import jax
import jax.numpy as jnp
from jax import lax
from jax.experimental import pallas as pl
from jax.experimental.pallas import tpu as pltpu
from jax.experimental.pallas import tpu_sc as plsc

NC = 2
NS = 16
LANES = 16


def _dotT(x, w):
    return lax.dot_general(x, w, (((1,), (1,)), ((), ())),
                           preferred_element_type=jnp.float32)


def _prep_body(f_ref, wl_ref, wi_ref, bsum_ref, z_ref, base_ref):
    f = f_ref[...]
    p1 = _dotT(f, wl_ref[...])
    p2 = _dotT(f * f, wi_ref[...])
    z_ref[...] = p1 + p2
    base_ref[...] = p1 + bsum_ref[...]


def _make_prep(n, d, r):
    return pl.pallas_call(
        _prep_body,
        grid=(n // r,),
        in_specs=[
            pl.BlockSpec((r, d), lambda i: (i, 0)),
            pl.BlockSpec((d, d), lambda i: (0, 0)),
            pl.BlockSpec((d, d), lambda i: (0, 0)),
            pl.BlockSpec((1, d), lambda i: (0, 0)),
        ],
        out_specs=[pl.BlockSpec((r, d), lambda i: (i, 0)),
                   pl.BlockSpec((r, d), lambda i: (i, 0))],
        out_shape=[jax.ShapeDtypeStruct((n, d), jnp.float32),
                   jax.ShapeDtypeStruct((n, d), jnp.float32)],
    )


def _make_spmm3(n, e, d, K, nchunk):
    nw = NC * NS
    epw = e // nw
    rpt = n // NS
    nzf = rpt // K
    zrem = rpt - nzf * K
    assert epw * nw == e and nchunk * K >= epw
    assert K % LANES == 0 and nchunk % 2 == 1 and nchunk >= 3
    npair = (nchunk - 1) // 2
    mesh = plsc.VectorSubcoreMesh(core_axis_name="c", subcore_axis_name="s")

    @functools.partial(
        pl.kernel,
        mesh=mesh,
        out_type=jax.ShapeDtypeStruct((6, n, d), jnp.float32),
        scratch_types=[
            pltpu.VMEM((K, d), jnp.float32),
            pltpu.VMEM((K, d), jnp.float32),
            pltpu.VMEM((3, K), jnp.float32),
            pltpu.VMEM((3, K), jnp.float32),
            pltpu.VMEM((K,), jnp.int32),
            pltpu.VMEM((K,), jnp.int32),
            pltpu.VMEM((K,), jnp.int32),
            pltpu.VMEM((K,), jnp.int32),
            pltpu.VMEM((K,), jnp.float32),
            pltpu.VMEM((K,), jnp.float32),
            pltpu.VMEM_SHARED((n, d), jnp.float32),
            pltpu.SemaphoreType.DMA,
            pltpu.SemaphoreType.DMA,
            pltpu.SemaphoreType.DMA,
            pltpu.SemaphoreType.DMA,
            pltpu.SemaphoreType.DMA,
            pltpu.SemaphoreType.DMA,
        ],
    )
    def spmm3(i0, i1, i2, z_hbm, out_hbm,
              bufa, bufb, cba, cbb, ridxa, ridxb, cidxa, cidxb, vba, vbb,
              acc, sga, sgb, sira, sirb, ssa, ssb):
        cid = lax.axis_index("c")
        sid = lax.axis_index("s")
        wid = cid * NS + sid

        def gstart(cidx, buf, sem):
            pltpu.async_copy(z_hbm.at[cidx], buf, sem)

        def gwait(cidx, buf, sem):
            pltpu.make_async_copy(z_hbm.at[cidx], buf, sem).wait()

        def sstart(buf, ridx, sem):
            pltpu.async_copy(buf, acc.at[ridx], sem, add=True)

        def swait(buf, ridx, sem):
            pltpu.make_async_copy(buf, acc.at[ridx], sem).wait()

        def cvt(cb, ridx, cidx, vb):
            for g in range(K // LANES):
                sl = pl.ds(g * LANES, LANES)
                ridx[sl] = lax.convert_element_type(cb[0, sl], jnp.int32)
                cidx[sl] = lax.convert_element_type(cb[1, sl], jnp.int32)
                vb[sl] = cb[2, sl]

        def mul(buf, vb):
            def group(g, ecarry):
                v16 = vb[pl.ds(g * LANES, LANES)]
                for l in range(LANES):
                    idx = jnp.full((LANES,), l, jnp.int32)
                    vspl = v16.at[idx].get(mode="promise_in_bounds",
                                           unique_indices=False)
                    for j in range(d // LANES):
                        sl = (g * LANES + l, pl.ds(j * LANES, LANES))
                        buf[sl] = buf[sl] * vspl
                return ecarry
            lax.fori_loop(0, K // LANES, group, 0)

        for b, iall in enumerate((i0, i1, i2)):
            ih = iall.at[wid]

            def istart(c, cb, sem):
                pltpu.async_copy(ih.at[c], cb, sem)

            def iwait(c, cb, sem):
                pltpu.make_async_copy(ih.at[c], cb, sem).wait()

            def zb(i, carry):
                for j in range(d // LANES):
                    bufa[i, pl.ds(j * LANES, LANES)] = jnp.zeros(
                        (LANES,), jnp.float32)
                return carry
            lax.fori_loop(0, K, zb, 0)
            zbase = sid * rpt
            for i in range(nzf):
                pltpu.async_copy(bufa, acc.at[pl.ds(zbase + i * K, K)], ssa)
            if zrem:
                pltpu.async_copy(bufa.at[pl.ds(0, zrem)],
                                 acc.at[pl.ds(zbase + nzf * K, zrem)], ssa)
            for i in range(nzf):
                pltpu.make_async_copy(
                    bufa, acc.at[pl.ds(zbase + i * K, K)], ssa).wait()
            if zrem:
                pltpu.make_async_copy(
                    bufa.at[pl.ds(0, zrem)],
                    acc.at[pl.ds(zbase + nzf * K, zrem)], ssa).wait()
            plsc.subcore_barrier()

            istart(0, cba, sira)
            istart(1, cbb, sirb)
            iwait(0, cba, sira)
            cvt(cba, ridxa, cidxa, vba)
            gstart(cidxa, bufa, sga)
            istart(2, cba, sira)
            iwait(1, cbb, sirb)
            cvt(cbb, ridxb, cidxb, vbb)
            gstart(cidxb, bufb, sgb)
            istart(3, cbb, sirb)
            gwait(cidxa, bufa, sga)
            mul(bufa, vba)
            sstart(bufa, ridxa, ssa)

            def pair(i, carry):
                cy = 1 + 2 * i

                gwait(cidxb, bufb, sgb)
                swait(bufa, ridxa, ssa)
                iwait(cy + 1, cba, sira)
                cvt(cba, ridxa, cidxa, vba)

                @pl.when(cy + 3 < nchunk)
                def _():
                    istart(cy + 3, cba, sira)

                gstart(cidxa, bufa, sga)
                mul(bufb, vbb)
                sstart(bufb, ridxb, ssb)

                gwait(cidxa, bufa, sga)
                swait(bufb, ridxb, ssb)

                @pl.when(cy + 2 < nchunk)
                def _():
                    iwait(cy + 2, cbb, sirb)
                    cvt(cbb, ridxb, cidxb, vbb)

                    @pl.when(cy + 4 < nchunk)
                    def _():
                        istart(cy + 4, cbb, sirb)

                    gstart(cidxb, bufb, sgb)

                mul(bufa, vba)
                sstart(bufa, ridxa, ssa)
                return carry
            lax.fori_loop(0, npair, pair, 0)
            swait(bufa, ridxa, ssa)
            plsc.subcore_barrier()

            @pl.when(sid == 0)
            def _():
                pltpu.sync_copy(acc, out_hbm.at[2 * b + cid])
            plsc.subcore_barrier()

    return spmm3


def _attn_body(sp_ref, base_ref, wa_ref, ba_ref, a_ref, out_ref):
    i = pl.program_id(0)

    @pl.when(i == 0)
    def _():
        out_ref[...] = jnp.zeros_like(out_ref)

    parts = []
    for b in range(3):
        msum = sp_ref[2 * b] + sp_ref[2 * b + 1] + base_ref[...]
        t = jnp.tanh(_dotT(msum, wa_ref[b]) + ba_ref[b][None, :])
        parts.append(jnp.sum(t * a_ref[b][None, :], axis=0))
    out_ref[...] = out_ref[...] + jnp.stack(parts, axis=0)


def _make_attn(n, d, r):
    return pl.pallas_call(
        _attn_body,
        grid=(n // r,),
        in_specs=[
            pl.BlockSpec((6, r, d), lambda i: (0, i, 0)),
            pl.BlockSpec((r, d), lambda i: (i, 0)),
            pl.BlockSpec((3, d, d), lambda i: (0, 0, 0)),
            pl.BlockSpec((3, d), lambda i: (0, 0)),
            pl.BlockSpec((3, d), lambda i: (0, 0)),
        ],
        out_specs=pl.BlockSpec((3, d), lambda i: (0, 0)),
        out_shape=jax.ShapeDtypeStruct((3, d), jnp.float32),
        compiler_params=pltpu.CompilerParams(
            dimension_semantics=("arbitrary",)),
    )


def _comb_body(sp_ref, base_ref, beta_ref, out_ref):
    be = beta_ref[...]
    s0 = sp_ref[0] + sp_ref[1]
    s1 = sp_ref[2] + sp_ref[3]
    s2 = sp_ref[4] + sp_ref[5]
    out_ref[...] = (base_ref[...] + be[0, 0] * s0 + be[0, 1] * s1
                    + be[0, 2] * s2)


def _make_comb(n, d, r):
    return pl.pallas_call(
        _comb_body,
        grid=(n // r,),
        in_specs=[
            pl.BlockSpec((6, r, d), lambda i: (0, i, 0)),
            pl.BlockSpec((r, d), lambda i: (i, 0)),
            pl.BlockSpec((1, d), lambda i: (0, 0)),
        ],
        out_specs=pl.BlockSpec((r, d), lambda i: (i, 0)),
        out_shape=jax.ShapeDtypeStruct((n, d), jnp.float32),
    )


def kernel(lap_indices, lap_values, trust_indices, trust_values,
           add_indices, add_values, features, W_lin, b_lin, W_iat, b_iat,
           W_am, b_am, W_aa, b_aa, W_at, b_at, a_main, a_add, a_trust):
    n, d = features.shape
    e = lap_values.shape[0]
    r = 2000

    bsum = (b_lin + b_iat).reshape(1, d)
    z, base = _make_prep(n, d, r)(features, W_lin, W_iat, bsum)

    f32 = jnp.float32
    nw = NC * NS
    kk = 112
    nch = 91
    epw = e // nw
    pad = nch * kk - epw

    def _pack(idx, vals):
        arr = jnp.stack([idx[0].astype(f32), idx[1].astype(f32), vals],
                        axis=0)
        arr = arr.reshape(3, nw, epw)
        arr = jnp.pad(arr, ((0, 0), (0, 0), (0, pad)))
        return arr.reshape(3, nw, nch, kk).transpose(1, 2, 0, 3)

    sp = _make_spmm3(n, e, d, kk, nch)(
        _pack(lap_indices, lap_values),
        _pack(add_indices, add_values),
        _pack(trust_indices, trust_values), z)

    wa = jnp.stack([W_am, W_aa, W_at])
    ba = jnp.stack([b_am, b_aa, b_at])
    av = jnp.stack([a_main[:, 0], a_add[:, 0], a_trust[:, 0]])
    colsums = _make_attn(n, d, r)(sp, base, wa, ba, av)
    w = colsums.sum(axis=1) / n
    beta = jax.nn.softmax(w)
    beta128 = jnp.zeros((1, d), jnp.float32).at[0, :3].set(beta)

    return _make_comb(n, d, r)(sp, base, beta128)


# speedup vs baseline: 1.0051x; 1.0051x over previous
import functools

import jax
import jax.numpy as jnp
from jax import lax
from jax.experimental import pallas as pl
from jax.experimental.pallas import tpu as pltpu
from jax.experimental.pallas import tpu_sc as plsc

NC = 2
NS = 16
LANES = 16


def _dotT(x, w):
    return lax.dot_general(x, w, (((1,), (1,)), ((), ())),
                           preferred_element_type=jnp.float32)


def _prep_body(f_ref, wl_ref, wi_ref, bsum_ref, z_ref, base_ref):
    f = f_ref[...]
    p1 = _dotT(f, wl_ref[...])
    p2 = _dotT(f * f, wi_ref[...])
    z_ref[...] = p1 + p2
    base_ref[...] = p1 + bsum_ref[...]


def _make_prep(n, d, r):
    return pl.pallas_call(
        _prep_body,
        grid=(n // r,),
        in_specs=[
            pl.BlockSpec((r, d), lambda i: (i, 0)),
            pl.BlockSpec((d, d), lambda i: (0, 0)),
            pl.BlockSpec((d, d), lambda i: (0, 0)),
            pl.BlockSpec((1, d), lambda i: (0, 0)),
        ],
        out_specs=[pl.BlockSpec((r, d), lambda i: (i, 0)),
                   pl.BlockSpec((r, d), lambda i: (i, 0))],
        out_shape=[jax.ShapeDtypeStruct((n, d), jnp.float32),
                   jax.ShapeDtypeStruct((n, d), jnp.float32)],
    )


def _make_spmm3(n, e, d, K, nchunk):
    nw = NC * NS
    epw = e // nw
    rpt = n // NS
    nzf = rpt // K
    zrem = rpt - nzf * K
    assert epw * nw == e and nchunk * K >= epw
    assert K % LANES == 0 and nchunk % 2 == 1 and nchunk >= 3
    npair = (nchunk - 1) // 2
    mesh = plsc.VectorSubcoreMesh(core_axis_name="c", subcore_axis_name="s")

    @functools.partial(
        pl.kernel,
        mesh=mesh,
        out_type=jax.ShapeDtypeStruct((6, n, d), jnp.float32),
        scratch_types=[
            pltpu.VMEM((K, d), jnp.float32),
            pltpu.VMEM((K, d), jnp.float32),
            pltpu.VMEM((3, K), jnp.float32),
            pltpu.VMEM((3, K), jnp.float32),
            pltpu.VMEM((K,), jnp.int32),
            pltpu.VMEM((K,), jnp.int32),
            pltpu.VMEM((K,), jnp.int32),
            pltpu.VMEM((K,), jnp.int32),
            pltpu.VMEM((K,), jnp.float32),
            pltpu.VMEM((K,), jnp.float32),
            pltpu.VMEM_SHARED((n, d), jnp.float32),
            pltpu.SemaphoreType.DMA,
            pltpu.SemaphoreType.DMA,
            pltpu.SemaphoreType.DMA,
            pltpu.SemaphoreType.DMA,
            pltpu.SemaphoreType.DMA,
            pltpu.SemaphoreType.DMA,
        ],
    )
    def spmm3(i0, i1, i2, z_hbm, out_hbm,
              bufa, bufb, cba, cbb, ridxa, ridxb, cidxa, cidxb, vba, vbb,
              acc, sga, sgb, sira, sirb, ssa, ssb):
        cid = lax.axis_index("c")
        sid = lax.axis_index("s")
        wid = cid * NS + sid

        def gstart(cidx, buf, sem):
            pltpu.async_copy(z_hbm.at[cidx], buf, sem)

        def gwait(cidx, buf, sem):
            pltpu.make_async_copy(z_hbm.at[cidx], buf, sem).wait()

        def sstart(buf, ridx, sem):
            pltpu.async_copy(buf, acc.at[ridx], sem, add=True)

        def swait(buf, ridx, sem):
            pltpu.make_async_copy(buf, acc.at[ridx], sem).wait()

        def cvt(cb, ridx, cidx, vb):
            for g in range(K // LANES):
                sl = pl.ds(g * LANES, LANES)
                ridx[sl] = lax.convert_element_type(cb[0, sl], jnp.int32)
                cidx[sl] = lax.convert_element_type(cb[1, sl], jnp.int32)
                vb[sl] = cb[2, sl]

        def mul(buf, vb):
            def group(g, ecarry):
                v16 = vb[pl.ds(g * LANES, LANES)]
                for l in range(LANES):
                    idx = jnp.full((LANES,), l, jnp.int32)
                    vspl = v16.at[idx].get(mode="promise_in_bounds",
                                           unique_indices=False)
                    for j in range(d // LANES):
                        sl = (g * LANES + l, pl.ds(j * LANES, LANES))
                        buf[sl] = buf[sl] * vspl
                return ecarry
            lax.fori_loop(0, K // LANES, group, 0)

        for b, iall in enumerate((i0, i1, i2)):
            ih = iall.at[wid]

            def istart(c, cb, sem):
                pltpu.async_copy(ih.at[c], cb, sem)

            def iwait(c, cb, sem):
                pltpu.make_async_copy(ih.at[c], cb, sem).wait()

            def zb(i, carry):
                for j in range(d // LANES):
                    bufa[i, pl.ds(j * LANES, LANES)] = jnp.zeros(
                        (LANES,), jnp.float32)
                return carry
            lax.fori_loop(0, K, zb, 0)
            zbase = sid * rpt
            for i in range(nzf):
                pltpu.async_copy(bufa, acc.at[pl.ds(zbase + i * K, K)], ssa)
            if zrem:
                pltpu.async_copy(bufa.at[pl.ds(0, zrem)],
                                 acc.at[pl.ds(zbase + nzf * K, zrem)], ssa)
            for i in range(nzf):
                pltpu.make_async_copy(
                    bufa, acc.at[pl.ds(zbase + i * K, K)], ssa).wait()
            if zrem:
                pltpu.make_async_copy(
                    bufa.at[pl.ds(0, zrem)],
                    acc.at[pl.ds(zbase + nzf * K, zrem)], ssa).wait()
            plsc.subcore_barrier()

            istart(0, cba, sira)
            istart(1, cbb, sirb)
            iwait(0, cba, sira)
            cvt(cba, ridxa, cidxa, vba)
            gstart(cidxa, bufa, sga)
            istart(2, cba, sira)
            iwait(1, cbb, sirb)
            cvt(cbb, ridxb, cidxb, vbb)
            gstart(cidxb, bufb, sgb)
            istart(3, cbb, sirb)
            gwait(cidxa, bufa, sga)
            mul(bufa, vba)
            sstart(bufa, ridxa, ssa)

            def pair(i, carry):
                cy = 1 + 2 * i

                gwait(cidxb, bufb, sgb)
                swait(bufa, ridxa, ssa)
                iwait(cy + 1, cba, sira)
                cvt(cba, ridxa, cidxa, vba)

                @pl.when(cy + 3 < nchunk)
                def _():
                    istart(cy + 3, cba, sira)

                gstart(cidxa, bufa, sga)
                mul(bufb, vbb)
                sstart(bufb, ridxb, ssb)

                gwait(cidxa, bufa, sga)
                swait(bufb, ridxb, ssb)

                @pl.when(cy + 2 < nchunk)
                def _():
                    iwait(cy + 2, cbb, sirb)
                    cvt(cbb, ridxb, cidxb, vbb)

                    @pl.when(cy + 4 < nchunk)
                    def _():
                        istart(cy + 4, cbb, sirb)

                    gstart(cidxb, bufb, sgb)

                mul(bufa, vba)
                sstart(bufa, ridxa, ssa)
                return carry
            lax.fori_loop(0, npair, pair, 0)
            swait(bufa, ridxa, ssa)
            plsc.subcore_barrier()

            @pl.when(sid == 0)
            def _():
                pltpu.sync_copy(acc, out_hbm.at[2 * b + cid])
            plsc.subcore_barrier()

    return spmm3


def _attn_body(sp_ref, base_ref, wa_ref, ba_ref, a_ref, out_ref):
    i = pl.program_id(0)

    @pl.when(i == 0)
    def _():
        out_ref[...] = jnp.zeros_like(out_ref)

    parts = []
    for b in range(3):
        msum = sp_ref[2 * b] + sp_ref[2 * b + 1] + base_ref[...]
        t = jnp.tanh(_dotT(msum, wa_ref[b]) + ba_ref[b][None, :])
        parts.append(jnp.sum(t * a_ref[b][None, :], axis=0))
    out_ref[...] = out_ref[...] + jnp.stack(parts, axis=0)


def _make_attn(n, d, r):
    return pl.pallas_call(
        _attn_body,
        grid=(n // r,),
        in_specs=[
            pl.BlockSpec((6, r, d), lambda i: (0, i, 0)),
            pl.BlockSpec((r, d), lambda i: (i, 0)),
            pl.BlockSpec((3, d, d), lambda i: (0, 0, 0)),
            pl.BlockSpec((3, d), lambda i: (0, 0)),
            pl.BlockSpec((3, d), lambda i: (0, 0)),
        ],
        out_specs=pl.BlockSpec((3, d), lambda i: (0, 0)),
        out_shape=jax.ShapeDtypeStruct((3, d), jnp.float32),
        compiler_params=pltpu.CompilerParams(
            dimension_semantics=("arbitrary",)),
    )


def _comb_body(sp_ref, base_ref, beta_ref, out_ref):
    be = beta_ref[...]
    s0 = sp_ref[0] + sp_ref[1]
    s1 = sp_ref[2] + sp_ref[3]
    s2 = sp_ref[4] + sp_ref[5]
    out_ref[...] = (base_ref[...] + be[0, 0] * s0 + be[0, 1] * s1
                    + be[0, 2] * s2)


def _make_comb(n, d, r):
    return pl.pallas_call(
        _comb_body,
        grid=(n // r,),
        in_specs=[
            pl.BlockSpec((6, r, d), lambda i: (0, i, 0)),
            pl.BlockSpec((r, d), lambda i: (i, 0)),
            pl.BlockSpec((1, d), lambda i: (0, 0)),
        ],
        out_specs=pl.BlockSpec((r, d), lambda i: (i, 0)),
        out_shape=jax.ShapeDtypeStruct((n, d), jnp.float32),
    )


def kernel(lap_indices, lap_values, trust_indices, trust_values,
           add_indices, add_values, features, W_lin, b_lin, W_iat, b_iat,
           W_am, b_am, W_aa, b_aa, W_at, b_at, a_main, a_add, a_trust):
    n, d = features.shape
    e = lap_values.shape[0]
    r = 2000

    bsum = (b_lin + b_iat).reshape(1, d)
    z, base = _make_prep(n, d, r)(features, W_lin, W_iat, bsum)

    f32 = jnp.float32
    nw = NC * NS
    kk = 112
    nch = 91
    epw = e // nw
    pad = nch * kk - epw

    def _pack(idx, vals):
        arr = jnp.stack([idx[0].astype(f32), idx[1].astype(f32), vals],
                        axis=0)
        arr = arr.reshape(3, nw, epw)
        rows_pad = jnp.broadcast_to(
            jnp.arange(pad, dtype=f32)[None, :], (nw, pad))[None]
        zeros_pad = jnp.zeros((2, nw, pad), f32)
        arr = jnp.concatenate(
            [arr, jnp.concatenate([rows_pad, zeros_pad], axis=0)], axis=2)
        return arr.reshape(3, nw, nch, kk).transpose(1, 2, 0, 3)

    sp = _make_spmm3(n, e, d, kk, nch)(
        _pack(lap_indices, lap_values),
        _pack(add_indices, add_values),
        _pack(trust_indices, trust_values), z)

    wa = jnp.stack([W_am, W_aa, W_at])
    ba = jnp.stack([b_am, b_aa, b_at])
    av = jnp.stack([a_main[:, 0], a_add[:, 0], a_trust[:, 0]])
    colsums = _make_attn(n, d, r)(sp, base, wa, ba, av)
    w = colsums.sum(axis=1) / n
    beta = jax.nn.softmax(w)
    beta128 = jnp.zeros((1, d), jnp.float32).at[0, :3].set(beta)

    return _make_comb(n, d, r)(sp, base, beta128)


# speedup vs baseline: 1.9161x; 1.9064x over previous
import functools

import jax
import jax.numpy as jnp
from jax import lax
from jax.experimental import pallas as pl
from jax.experimental.pallas import tpu as pltpu
from jax.experimental.pallas import tpu_sc as plsc

NC = 2
NS = 16
LANES = 16


def _dotT(x, w):
    return lax.dot_general(x, w, (((1,), (1,)), ((), ())),
                           preferred_element_type=jnp.float32)


def _prep_body(f_ref, wl_ref, wi_ref, bsum_ref, z_ref, base_ref):
    f = f_ref[...]
    p1 = _dotT(f, wl_ref[...])
    p2 = _dotT(f * f, wi_ref[...])
    z_ref[...] = p1 + p2
    base_ref[...] = p1 + bsum_ref[...]


def _make_prep(n, d, r):
    return pl.pallas_call(
        _prep_body,
        grid=(n // r,),
        in_specs=[
            pl.BlockSpec((r, d), lambda i: (i, 0)),
            pl.BlockSpec((d, d), lambda i: (0, 0)),
            pl.BlockSpec((d, d), lambda i: (0, 0)),
            pl.BlockSpec((1, d), lambda i: (0, 0)),
        ],
        out_specs=[pl.BlockSpec((r, d), lambda i: (i, 0)),
                   pl.BlockSpec((r, d), lambda i: (i, 0))],
        out_shape=[jax.ShapeDtypeStruct((n, d), jnp.float32),
                   jax.ShapeDtypeStruct((n, d), jnp.float32)],
    )


def _make_spmm3(n, e, d, K, nchunk):
    nw = NC * NS
    epw = e // nw
    rpt = n // NS
    nzf = rpt // K
    zrem = rpt - nzf * K
    assert epw * nw == e and nchunk * K >= epw
    assert K % LANES == 0 and nchunk % 2 == 1 and nchunk >= 3
    npair = (nchunk - 1) // 2
    mesh = plsc.VectorSubcoreMesh(core_axis_name="c", subcore_axis_name="s")

    @functools.partial(
        pl.kernel,
        mesh=mesh,
        out_type=jax.ShapeDtypeStruct((6, n, d), jnp.float32),
        scratch_types=[
            pltpu.VMEM((K, d), jnp.float32),
            pltpu.VMEM((K, d), jnp.float32),
            pltpu.VMEM((3, K), jnp.float32),
            pltpu.VMEM((3, K), jnp.float32),
            pltpu.VMEM((K,), jnp.int32),
            pltpu.VMEM((K,), jnp.int32),
            pltpu.VMEM((K,), jnp.int32),
            pltpu.VMEM((K,), jnp.int32),
            pltpu.VMEM((K,), jnp.float32),
            pltpu.VMEM((K,), jnp.float32),
            pltpu.VMEM_SHARED((n, d), jnp.float32),
            pltpu.SemaphoreType.DMA,
            pltpu.SemaphoreType.DMA,
            pltpu.SemaphoreType.DMA,
            pltpu.SemaphoreType.DMA,
            pltpu.SemaphoreType.DMA,
            pltpu.SemaphoreType.DMA,
        ],
    )
    def spmm3(i0, i1, i2, z_hbm, out_hbm,
              bufa, bufb, cba, cbb, ridxa, ridxb, cidxa, cidxb, vba, vbb,
              acc, sga, sgb, sira, sirb, ssa, ssb):
        cid = lax.axis_index("c")
        sid = lax.axis_index("s")
        wid = cid * NS + sid

        def gstart(cidx, buf, sem):
            pltpu.async_copy(z_hbm.at[cidx], buf, sem)

        def gwait(cidx, buf, sem):
            pltpu.make_async_copy(z_hbm.at[cidx], buf, sem).wait()

        def sstart(buf, ridx, sem):
            pltpu.async_copy(buf, acc.at[ridx], sem, add=True)

        def swait(buf, ridx, sem):
            pltpu.make_async_copy(buf, acc.at[ridx], sem).wait()

        def cvt(cb, ridx, cidx, vb):
            for g in range(K // LANES):
                sl = pl.ds(g * LANES, LANES)
                ridx[sl] = lax.convert_element_type(cb[0, sl], jnp.int32)
                cidx[sl] = lax.convert_element_type(cb[1, sl], jnp.int32)
                vb[sl] = cb[2, sl]

        def mul(buf, vb):
            def group(g, ecarry):
                v16 = vb[pl.ds(g * LANES, LANES)]
                for l in range(LANES):
                    idx = jnp.full((LANES,), l, jnp.int32)
                    vspl = v16.at[idx].get(mode="promise_in_bounds",
                                           unique_indices=False)
                    for j in range(d // LANES):
                        sl = (g * LANES + l, pl.ds(j * LANES, LANES))
                        buf[sl] = buf[sl] * vspl
                return ecarry
            lax.fori_loop(0, K // LANES, group, 0)

        for b, iall in enumerate((i0, i1, i2)):
            ih = iall.at[wid]

            def istart(c, cb, sem):
                pltpu.async_copy(ih.at[c], cb, sem)

            def iwait(c, cb, sem):
                pltpu.make_async_copy(ih.at[c], cb, sem).wait()

            def zb(i, carry):
                for j in range(d // LANES):
                    bufa[i, pl.ds(j * LANES, LANES)] = jnp.zeros(
                        (LANES,), jnp.float32)
                return carry
            lax.fori_loop(0, K, zb, 0)
            zbase = sid * rpt
            for i in range(nzf):
                pltpu.async_copy(bufa, acc.at[pl.ds(zbase + i * K, K)], ssa)
            if zrem:
                pltpu.async_copy(bufa.at[pl.ds(0, zrem)],
                                 acc.at[pl.ds(zbase + nzf * K, zrem)], ssa)
            for i in range(nzf):
                pltpu.make_async_copy(
                    bufa, acc.at[pl.ds(zbase + i * K, K)], ssa).wait()
            if zrem:
                pltpu.make_async_copy(
                    bufa.at[pl.ds(0, zrem)],
                    acc.at[pl.ds(zbase + nzf * K, zrem)], ssa).wait()
            plsc.subcore_barrier()

            istart(0, cba, sira)
            istart(1, cbb, sirb)
            iwait(0, cba, sira)
            cvt(cba, ridxa, cidxa, vba)
            gstart(cidxa, bufa, sga)
            istart(2, cba, sira)
            iwait(1, cbb, sirb)
            cvt(cbb, ridxb, cidxb, vbb)
            gstart(cidxb, bufb, sgb)
            istart(3, cbb, sirb)
            gwait(cidxa, bufa, sga)
            mul(bufa, vba)
            sstart(bufa, ridxa, ssa)

            def pair(i, carry):
                cy = 1 + 2 * i

                gwait(cidxb, bufb, sgb)
                swait(bufa, ridxa, ssa)
                iwait(cy + 1, cba, sira)
                cvt(cba, ridxa, cidxa, vba)

                @pl.when(cy + 3 < nchunk)
                def _():
                    istart(cy + 3, cba, sira)

                gstart(cidxa, bufa, sga)
                mul(bufb, vbb)
                sstart(bufb, ridxb, ssb)

                gwait(cidxa, bufa, sga)
                swait(bufb, ridxb, ssb)

                @pl.when(cy + 2 < nchunk)
                def _():
                    iwait(cy + 2, cbb, sirb)
                    cvt(cbb, ridxb, cidxb, vbb)

                    @pl.when(cy + 4 < nchunk)
                    def _():
                        istart(cy + 4, cbb, sirb)

                    gstart(cidxb, bufb, sgb)

                mul(bufa, vba)
                sstart(bufa, ridxa, ssa)
                return carry
            lax.fori_loop(0, npair, pair, 0)
            swait(bufa, ridxa, ssa)
            plsc.subcore_barrier()

            @pl.when(sid == 0)
            def _():
                pltpu.sync_copy(acc, out_hbm.at[2 * b + cid])
            plsc.subcore_barrier()

    return spmm3


def _attn_body(sp_ref, base_ref, wa_ref, ba_ref, a_ref, out_ref):
    i = pl.program_id(0)

    @pl.when(i == 0)
    def _():
        out_ref[...] = jnp.zeros_like(out_ref)

    parts = []
    for b in range(3):
        msum = sp_ref[2 * b] + sp_ref[2 * b + 1] + base_ref[...]
        t = jnp.tanh(_dotT(msum, wa_ref[b]) + ba_ref[b][None, :])
        parts.append(jnp.sum(t * a_ref[b][None, :], axis=0))
    out_ref[...] = out_ref[...] + jnp.stack(parts, axis=0)


def _make_attn(n, d, r):
    return pl.pallas_call(
        _attn_body,
        grid=(n // r,),
        in_specs=[
            pl.BlockSpec((6, r, d), lambda i: (0, i, 0)),
            pl.BlockSpec((r, d), lambda i: (i, 0)),
            pl.BlockSpec((3, d, d), lambda i: (0, 0, 0)),
            pl.BlockSpec((3, d), lambda i: (0, 0)),
            pl.BlockSpec((3, d), lambda i: (0, 0)),
        ],
        out_specs=pl.BlockSpec((3, d), lambda i: (0, 0)),
        out_shape=jax.ShapeDtypeStruct((3, d), jnp.float32),
        compiler_params=pltpu.CompilerParams(
            dimension_semantics=("arbitrary",)),
    )


def _comb_body(sp_ref, base_ref, beta_ref, out_ref):
    be = beta_ref[...]
    s0 = sp_ref[0] + sp_ref[1]
    s1 = sp_ref[2] + sp_ref[3]
    s2 = sp_ref[4] + sp_ref[5]
    out_ref[...] = (base_ref[...] + be[0, 0] * s0 + be[0, 1] * s1
                    + be[0, 2] * s2)


def _make_comb(n, d, r):
    return pl.pallas_call(
        _comb_body,
        grid=(n // r,),
        in_specs=[
            pl.BlockSpec((6, r, d), lambda i: (0, i, 0)),
            pl.BlockSpec((r, d), lambda i: (i, 0)),
            pl.BlockSpec((1, d), lambda i: (0, 0)),
        ],
        out_specs=pl.BlockSpec((r, d), lambda i: (i, 0)),
        out_shape=jax.ShapeDtypeStruct((n, d), jnp.float32),
    )


def kernel(lap_indices, lap_values, trust_indices, trust_values,
           add_indices, add_values, features, W_lin, b_lin, W_iat, b_iat,
           W_am, b_am, W_aa, b_aa, W_at, b_at, a_main, a_add, a_trust):
    n, d = features.shape
    e = lap_values.shape[0]
    r = 2000

    bsum = (b_lin + b_iat).reshape(1, d)
    z, base = _make_prep(n, d, r)(features, W_lin, W_iat, bsum)

    f32 = jnp.float32
    nw = NC * NS
    kk = 80
    nch = 125
    epw = e // nw
    pad = nch * kk - epw

    def _pack(idx, vals):
        arr = jnp.stack([idx[0].astype(f32), idx[1].astype(f32), vals],
                        axis=0)
        arr = arr.reshape(3, nw, epw)
        rows_pad = jnp.broadcast_to(
            jnp.arange(pad, dtype=f32)[None, :], (nw, pad))[None]
        zeros_pad = jnp.zeros((2, nw, pad), f32)
        arr = jnp.concatenate(
            [arr, jnp.concatenate([rows_pad, zeros_pad], axis=0)], axis=2)
        return arr.reshape(3, nw, nch, kk).transpose(1, 2, 0, 3)

    sp = _make_spmm3(n, e, d, kk, nch)(
        _pack(lap_indices, lap_values),
        _pack(add_indices, add_values),
        _pack(trust_indices, trust_values), z)

    wa = jnp.stack([W_am, W_aa, W_at])
    ba = jnp.stack([b_am, b_aa, b_at])
    av = jnp.stack([a_main[:, 0], a_add[:, 0], a_trust[:, 0]])
    colsums = _make_attn(n, d, r)(sp, base, wa, ba, av)
    w = colsums.sum(axis=1) / n
    beta = jax.nn.softmax(w)
    beta128 = jnp.zeros((1, d), jnp.float32).at[0, :3].set(beta)

    return _make_comb(n, d, r)(sp, base, beta128)
